# baseline (speedup 1.0000x reference)
import functools

import jax
import jax.numpy as jnp
from jax import lax
from jax.experimental import pallas as pl
from jax.experimental.pallas import tpu as pltpu
from jax.experimental.pallas import tpu_sc as plsc

_R = 32
_RC = 8
_L = 16
_NT = 1280


def _tc_body(tok_ref, feat_ref, out_ref, *, hw, c, topk):
    tok = tok_ref[...]
    feat = feat_ref[...]
    scale = c ** -0.5

    parts = []
    for q in range(_R // _RC):
        fq = feat[q * _RC:(q + 1) * _RC]
        tq = tok[q * _RC:(q + 1) * _RC]
        f2 = fq.reshape(_RC * hw, c)
        sf = jax.lax.dot_general(
            f2, tq, (((1,), (1,)), ((), ())),
            precision=jax.lax.Precision.DEFAULT,
            preferred_element_type=jnp.float32)
        s3 = sf.reshape(_RC, hw, _RC)
        rr = jax.lax.broadcasted_iota(jnp.int32, s3.shape, 0)
        ll = jax.lax.broadcasted_iota(jnp.int32, s3.shape, 2)
        parts.append(jnp.sum(jnp.where(rr == ll, s3, 0.0), axis=-1))
    s = jnp.concatenate(parts, axis=0) * scale
    m = jnp.max(s, axis=-1, keepdims=True)

    bits = jax.lax.bitcast_convert_type(s, jnp.int32)
    key = jnp.where(bits >= 0, bits, bits ^ jnp.int32(0x7FFFFFFF))
    keyT = jnp.transpose(key)
    lo = jnp.min(keyT, axis=0, keepdims=True)
    hi = jnp.max(keyT, axis=0, keepdims=True) + 1
    for _ in range(32):
        mid = (lo >> 1) + (hi >> 1) + (lo & hi & 1)
        cnt = jnp.sum((keyT >= mid).astype(jnp.int32), axis=0,
                      keepdims=True)
        ge = cnt >= topk
        lo = jnp.where(ge, mid, lo)
        hi = jnp.where(ge, hi, mid)
    t = jnp.transpose(lo)

    w = jnp.where(key >= t, jnp.exp(s - m), 0.0)
    denom = jnp.sum(w, axis=-1, keepdims=True)
    out = jnp.sum(feat * w[:, :, None], axis=1)
    out_ref[...] = out / denom


def _sc_body(tok_hbm, feat_hbm, out_hbm, tok_v, fbuf, s_v, k_v, o_v,
             *, hw, c, topk, rpw, nc, base0):
    wid = lax.axis_index("s") * nc + lax.axis_index("c")
    base = base0 + wid * rpw
    pltpu.sync_copy(tok_hbm.at[pl.ds(base, rpw)], tok_v)
    nl = c // _L
    nq = hw // _L
    scale = c ** -0.5
    lane0 = lax.iota(jnp.int32, _L) == 0

    def row_body(g, _):
        row = base + g
        pltpu.sync_copy(feat_hbm.at[row], fbuf)
        tvecs = [tok_v[g, pl.ds(l * _L, _L)] for l in range(nl)]

        def score_j(j, _):
            acc = fbuf[j, pl.ds(0, _L)] * tvecs[0]
            for l in range(1, nl):
                acc = acc + fbuf[j, pl.ds(l * _L, _L)] * tvecs[l]
            val = jnp.sum(acc) * scale
            plsc.store_scatter(s_v, [jnp.full((_L,), j, jnp.int32)],
                               jnp.full((_L,), val, jnp.float32),
                               mask=lane0)
            return 0

        lax.fori_loop(0, hw, score_j, 0)

        def max_q(q, vmx):
            return jnp.maximum(vmx, s_v[pl.ds(q * _L, _L)])

        vmx = lax.fori_loop(1, nq, max_q, s_v[pl.ds(0, _L)])
        m = jnp.max(vmx)

        def key_q(q, _):
            sv = s_v[pl.ds(q * _L, _L)]
            bb = lax.bitcast_convert_type(sv, jnp.int32)
            k_v[pl.ds(q * _L, _L)] = jnp.where(
                bb >= 0, bb, bb ^ jnp.int32(0x7FFFFFFF))
            return 0

        lax.fori_loop(0, nq, key_q, 0)

        def minmax_q(q, mm):
            kv = k_v[pl.ds(q * _L, _L)]
            return (jnp.minimum(mm[0], kv), jnp.maximum(mm[1], kv))

        kv0 = k_v[pl.ds(0, _L)]
        vmn, vmx2 = lax.fori_loop(1, nq, minmax_q, (kv0, kv0))
        lo = jnp.min(vmn)
        hi = jnp.max(vmx2) + 1

        def bis(i, lohi):
            lo, hi = lohi
            mid = (lo >> 1) + (hi >> 1) + (lo & hi & 1)

            def cnt_q(q, cv):
                kv = k_v[pl.ds(q * _L, _L)]
                return cv + plsc.all_reduce_population_count(kv >= mid)

            cv = lax.fori_loop(0, nq, cnt_q, jnp.zeros((_L,), jnp.int32))
            cnt = jnp.max(cv)
            ge = cnt >= topk
            return (jnp.where(ge, mid, lo), jnp.where(ge, hi, mid))

        t, _hi = lax.fori_loop(0, 32, bis, (lo, hi))

        def w_q(q, acc):
            sv = s_v[pl.ds(q * _L, _L)]
            kv = k_v[pl.ds(q * _L, _L)]
            wv = jnp.where(kv >= t, jnp.exp(sv - m), 0.0)
            s_v[pl.ds(q * _L, _L)] = wv
            return acc + wv

        wacc = lax.fori_loop(0, nq, w_q, jnp.zeros((_L,), jnp.float32))
        inv = 1.0 / jnp.full((_L,), jnp.sum(wacc), jnp.float32)

        def comb_q(q, accs):
            wv = s_v[pl.ds(q * _L, _L)]
            accs = list(accs)
            for u in range(_L):
                j = q * _L + u
                wj = wv[u]
                for l in range(nl):
                    accs[l] = accs[l] + wj * fbuf[j, pl.ds(l * _L, _L)]
            return tuple(accs)

        accs = lax.fori_loop(
            0, nq, comb_q,
            tuple(jnp.zeros((_L,), jnp.float32) for _ in range(nl)))
        for l in range(nl):
            o_v[pl.ds(l * _L, _L)] = accs[l] * inv
        pltpu.sync_copy(o_v, out_hbm.at[row - base0])
        return 0

    lax.fori_loop(0, rpw, row_body, 0)


def kernel(token, feature):
    b, n, k, c = token.shape
    hw = feature.shape[3]
    nrows = b * n * k
    topk = int(hw * 0.5)
    tok = token.reshape(nrows, c)
    feat = feature.reshape(nrows, hw, c)

    info = plsc.get_sparse_core_info()
    nw = info.num_cores * info.num_subcores
    nt = _NT
    nsc = nrows - nt
    if nsc % nw != 0 or nt % _R != 0:
        nt = nrows
        nsc = 0

    tc_body = functools.partial(_tc_body, hw=hw, c=c, topk=topk)
    out_tc = pl.pallas_call(
        tc_body,
        grid=(nt // _R,),
        in_specs=[
            pl.BlockSpec((_R, c), lambda i: (i, 0)),
            pl.BlockSpec((_R, hw, c), lambda i: (i, 0, 0)),
        ],
        out_specs=pl.BlockSpec((_R, c), lambda i: (i, 0)),
        out_shape=jax.ShapeDtypeStruct((nt, c), jnp.float32),
    )(tok, feat)

    if nsc == 0:
        return out_tc.reshape(b, n, k, c)

    sc_body = functools.partial(_sc_body, hw=hw, c=c, topk=topk,
                                rpw=nsc // nw, nc=info.num_cores, base0=nt)
    sc_fn = pl.kernel(
        sc_body,
        mesh=plsc.VectorSubcoreMesh(core_axis_name="c",
                                    subcore_axis_name="s"),
        out_type=jax.ShapeDtypeStruct((nsc, c), jnp.float32),
        scratch_types=[
            pltpu.VMEM((nsc // nw, c), jnp.float32),
            pltpu.VMEM((hw, c), jnp.float32),
            pltpu.VMEM((hw,), jnp.float32),
            pltpu.VMEM((hw,), jnp.int32),
            pltpu.VMEM((c,), jnp.float32),
        ],
        compiler_params=pltpu.CompilerParams(needs_layout_passes=False),
    )
    out_sc = sc_fn(tok, feat)

    out = jnp.concatenate([out_tc, out_sc], axis=0)
    return out.reshape(b, n, k, c)


# speedup vs baseline: 1.2029x; 1.2029x over previous
import functools

import jax
import jax.numpy as jnp
from jax import lax
from jax.experimental import pallas as pl
from jax.experimental.pallas import tpu as pltpu
from jax.experimental.pallas import tpu_sc as plsc

_R = 32
_RC = 8
_L = 16
_NT = 1536


def _tc_body(tok_ref, feat_ref, out_ref, *, hw, c, topk):
    tok = tok_ref[...]
    feat = feat_ref[...]
    scale = c ** -0.5

    parts = []
    for q in range(_R // _RC):
        fq = feat[q * _RC:(q + 1) * _RC]
        tq = tok[q * _RC:(q + 1) * _RC]
        f2 = fq.reshape(_RC * hw, c)
        sf = jax.lax.dot_general(
            f2, tq, (((1,), (1,)), ((), ())),
            precision=jax.lax.Precision.DEFAULT,
            preferred_element_type=jnp.float32)
        s3 = sf.reshape(_RC, hw, _RC)
        rr = jax.lax.broadcasted_iota(jnp.int32, s3.shape, 0)
        ll = jax.lax.broadcasted_iota(jnp.int32, s3.shape, 2)
        parts.append(jnp.sum(jnp.where(rr == ll, s3, 0.0), axis=-1))
    s = jnp.concatenate(parts, axis=0) * scale
    m = jnp.max(s, axis=-1, keepdims=True)

    bits = jax.lax.bitcast_convert_type(s, jnp.int32)
    key = jnp.where(bits >= 0, bits, bits ^ jnp.int32(0x7FFFFFFF))
    keyT = jnp.transpose(key)
    lo = jnp.min(keyT, axis=0, keepdims=True)
    hi = jnp.max(keyT, axis=0, keepdims=True) + 1
    for _ in range(32):
        mid = (lo >> 1) + (hi >> 1) + (lo & hi & 1)
        cnt = jnp.sum((keyT >= mid).astype(jnp.int32), axis=0,
                      keepdims=True)
        ge = cnt >= topk
        lo = jnp.where(ge, mid, lo)
        hi = jnp.where(ge, hi, mid)
    t = jnp.transpose(lo)

    w = jnp.where(key >= t, jnp.exp(s - m), 0.0)
    denom = jnp.sum(w, axis=-1, keepdims=True)
    out = jnp.sum(feat * w[:, :, None], axis=1)
    out_ref[...] = out / denom


def _sc_body(tok_hbm, feat_hbm, out_hbm, tok_v, fbuf, s_v, k_v, o_v,
             *, hw, c, topk, rpw, nc, base0):
    wid = lax.axis_index("s") * nc + lax.axis_index("c")
    base = base0 + wid * rpw
    pltpu.sync_copy(tok_hbm.at[pl.ds(base, rpw)], tok_v)
    nl = c // _L
    nq = hw // _L
    scale = c ** -0.5
    lane0 = lax.iota(jnp.int32, _L) == 0

    def row_body(g, _):
        row = base + g
        pltpu.sync_copy(feat_hbm.at[row], fbuf)
        tvecs = [tok_v[g, pl.ds(l * _L, _L)] for l in range(nl)]

        def score_j(j, _):
            acc = fbuf[j, pl.ds(0, _L)] * tvecs[0]
            for l in range(1, nl):
                acc = acc + fbuf[j, pl.ds(l * _L, _L)] * tvecs[l]
            val = jnp.sum(acc) * scale
            plsc.store_scatter(s_v, [jnp.full((_L,), j, jnp.int32)],
                               jnp.full((_L,), val, jnp.float32),
                               mask=lane0)
            return 0

        lax.fori_loop(0, hw, score_j, 0)

        def max_q(q, vmx):
            return jnp.maximum(vmx, s_v[pl.ds(q * _L, _L)])

        vmx = lax.fori_loop(1, nq, max_q, s_v[pl.ds(0, _L)])
        m = jnp.max(vmx)

        def key_q(q, _):
            sv = s_v[pl.ds(q * _L, _L)]
            bb = lax.bitcast_convert_type(sv, jnp.int32)
            k_v[pl.ds(q * _L, _L)] = jnp.where(
                bb >= 0, bb, bb ^ jnp.int32(0x7FFFFFFF))
            return 0

        lax.fori_loop(0, nq, key_q, 0)

        def minmax_q(q, mm):
            kv = k_v[pl.ds(q * _L, _L)]
            return (jnp.minimum(mm[0], kv), jnp.maximum(mm[1], kv))

        kv0 = k_v[pl.ds(0, _L)]
        vmn, vmx2 = lax.fori_loop(1, nq, minmax_q, (kv0, kv0))
        lo = jnp.min(vmn)
        hi = jnp.max(vmx2) + 1

        def bis(i, lohi):
            lo, hi = lohi
            mid = (lo >> 1) + (hi >> 1) + (lo & hi & 1)

            def cnt_q(q, cv):
                kv = k_v[pl.ds(q * _L, _L)]
                return cv + plsc.all_reduce_population_count(kv >= mid)

            cv = lax.fori_loop(0, nq, cnt_q, jnp.zeros((_L,), jnp.int32))
            cnt = jnp.max(cv)
            ge = cnt >= topk
            return (jnp.where(ge, mid, lo), jnp.where(ge, hi, mid))

        t, _hi = lax.fori_loop(0, 32, bis, (lo, hi))

        def w_q(q, acc):
            sv = s_v[pl.ds(q * _L, _L)]
            kv = k_v[pl.ds(q * _L, _L)]
            wv = jnp.where(kv >= t, jnp.exp(sv - m), 0.0)
            s_v[pl.ds(q * _L, _L)] = wv
            return acc + wv

        wacc = lax.fori_loop(0, nq, w_q, jnp.zeros((_L,), jnp.float32))
        inv = 1.0 / jnp.full((_L,), jnp.sum(wacc), jnp.float32)

        def comb_q(q, accs):
            wv = s_v[pl.ds(q * _L, _L)]
            accs = list(accs)
            for u in range(_L):
                j = q * _L + u
                wj = wv[u]
                for l in range(nl):
                    accs[l] = accs[l] + wj * fbuf[j, pl.ds(l * _L, _L)]
            return tuple(accs)

        accs = lax.fori_loop(
            0, nq, comb_q,
            tuple(jnp.zeros((_L,), jnp.float32) for _ in range(nl)))
        for l in range(nl):
            o_v[pl.ds(l * _L, _L)] = accs[l] * inv
        pltpu.sync_copy(o_v, out_hbm.at[row - base0])
        return 0

    lax.fori_loop(0, rpw, row_body, 0)


def kernel(token, feature):
    b, n, k, c = token.shape
    hw = feature.shape[3]
    nrows = b * n * k
    topk = int(hw * 0.5)
    tok = token.reshape(nrows, c)
    feat = feature.reshape(nrows, hw, c)

    info = plsc.get_sparse_core_info()
    nw = info.num_cores * info.num_subcores
    nt = _NT
    nsc = nrows - nt
    if nsc % nw != 0 or nt % _R != 0:
        nt = nrows
        nsc = 0

    tc_body = functools.partial(_tc_body, hw=hw, c=c, topk=topk)
    out_tc = pl.pallas_call(
        tc_body,
        grid=(nt // _R,),
        in_specs=[
            pl.BlockSpec((_R, c), lambda i: (i, 0)),
            pl.BlockSpec((_R, hw, c), lambda i: (i, 0, 0)),
        ],
        out_specs=pl.BlockSpec((_R, c), lambda i: (i, 0)),
        out_shape=jax.ShapeDtypeStruct((nt, c), jnp.float32),
    )(tok, feat)

    if nsc == 0:
        return out_tc.reshape(b, n, k, c)

    sc_body = functools.partial(_sc_body, hw=hw, c=c, topk=topk,
                                rpw=nsc // nw, nc=info.num_cores, base0=nt)
    sc_fn = pl.kernel(
        sc_body,
        mesh=plsc.VectorSubcoreMesh(core_axis_name="c",
                                    subcore_axis_name="s"),
        out_type=jax.ShapeDtypeStruct((nsc, c), jnp.float32),
        scratch_types=[
            pltpu.VMEM((nsc // nw, c), jnp.float32),
            pltpu.VMEM((hw, c), jnp.float32),
            pltpu.VMEM((hw,), jnp.float32),
            pltpu.VMEM((hw,), jnp.int32),
            pltpu.VMEM((c,), jnp.float32),
        ],
        compiler_params=pltpu.CompilerParams(needs_layout_passes=False),
    )
    out_sc = sc_fn(tok, feat)

    out = jnp.concatenate([out_tc, out_sc], axis=0)
    return out.reshape(b, n, k, c)
